# baseline (speedup 1.0000x reference)
import functools

import jax
import jax.numpy as jnp
from jax.experimental import pallas as pl


def _mm_body(a_ref, b_ref, o_ref):
    o_ref[:] = jnp.dot(a_ref[:], b_ref[:], preferred_element_type=jnp.float32)


def _matmul(a, b):
    m, k = a.shape
    k2, n = b.shape
    assert k == k2
    mb = 2000 if m % 2000 == 0 else m
    grid = m // mb
    return pl.pallas_call(
        _mm_body,
        grid=(grid,),
        in_specs=[
            pl.BlockSpec((mb, k), lambda i: (i, 0)),
            pl.BlockSpec((k, n), lambda i: (0, 0)),
        ],
        out_specs=pl.BlockSpec((mb, n), lambda i: (i, 0)),
        out_shape=jax.ShapeDtypeStruct((m, n), jnp.float32),
    )(a, b)


def _layer(x, src, dst, node_type, edge_attr, edge_type, p, concat):
    n_nodes = x.shape[0]
    heads, out_ch = p["a_src"].shape
    in_ch = p["W"].shape[0]
    d_edge = p["W_e"].shape[0]
    num_types = p["et_emb"].shape[0]

    w3 = p["W"].reshape(in_ch, heads, out_ch)
    we3 = p["W_e"].reshape(d_edge, heads, out_ch)
    et3 = p["et_emb"].reshape(num_types, heads, out_ch)

    a_src_w = jnp.einsum("ihc,hc->ih", w3, p["a_src"])
    a_dst_w = jnp.einsum("ihc,hc->ih", w3, p["a_dst"])
    w_ae = jnp.einsum("dhc,hc->dh", we3, p["a_edge"])
    t_a = jnp.einsum("thc,hc->th", et3, p["a_edge"])

    xt = x + jnp.take(p["nt_emb"], node_type, axis=0)

    wcat = jnp.concatenate([p["W"], a_src_w, a_dst_w], axis=1)
    hcat = _matmul(xt, wcat)
    s_src = hcat[:, heads * out_ch : heads * out_ch + heads]
    s_dst = hcat[:, heads * out_ch + heads :]

    base = _matmul(edge_attr, w_ae) + jnp.take(t_a, edge_type, axis=0)

    alpha = jnp.take(s_src, src, axis=0) + jnp.take(s_dst, dst, axis=0) + base
    alpha = jnp.where(alpha >= 0, alpha, 0.2 * alpha)
    ex = jnp.exp(alpha)

    denom = jax.ops.segment_sum(ex, dst, num_segments=n_nodes)
    xs = jnp.take(xt, src, axis=0)
    g_acc = jax.ops.segment_sum(ex[:, :, None] * xs[:, None, :], dst, num_segments=n_nodes)
    f_acc = jax.ops.segment_sum(ex[:, :, None] * edge_attr[:, None, :], dst, num_segments=n_nodes)
    one_hot = jax.nn.one_hot(edge_type, num_types, dtype=jnp.float32)
    t_acc = jax.ops.segment_sum(ex[:, :, None] * one_hot[:, None, :], dst, num_segments=n_nodes)

    u = (
        jnp.einsum("nhi,ihc->nhc", g_acc, w3)
        + jnp.einsum("nhd,dhc->nhc", f_acc, we3)
        + jnp.einsum("nht,thc->nhc", t_acc, et3)
    )
    out = u / (denom[:, :, None] + 1e-16)
    if concat:
        out = out.reshape(n_nodes, heads * out_ch)
    else:
        out = out.mean(axis=1)
    out = out + p["bias"]
    return jnp.where(out > 0, out, jnp.expm1(out))


def kernel(x, edge_index, node_type, edge_attr, edge_type, params1, params2):
    src = edge_index[0]
    dst = edge_index[1]
    h = _layer(x, src, dst, node_type, edge_attr, edge_type, params1, True)
    h = _layer(h, src, dst, node_type, edge_attr, edge_type, params2, False)
    return h


# speedup vs baseline: 12.2110x; 12.2110x over previous
import functools

import jax
import jax.numpy as jnp
from jax import lax
from jax.experimental import pallas as pl
from jax.experimental.pallas import tpu as pltpu
from jax.experimental.pallas import tpu_sc as plsc

_NW = 32
_BP = 128


def _mm_body(a_ref, b_ref, o_ref):
    o_ref[:] = jnp.dot(a_ref[:], b_ref[:], preferred_element_type=jnp.float32)


def _matmul(a, b):
    m, k = a.shape
    k2, n = b.shape
    assert k == k2
    mb = 2000 if m % 2000 == 0 else m
    grid = m // mb
    return pl.pallas_call(
        _mm_body,
        grid=(grid,),
        in_specs=[
            pl.BlockSpec((mb, k), lambda i: (i, 0)),
            pl.BlockSpec((k, n), lambda i: (0, 0)),
        ],
        out_specs=pl.BlockSpec((mb, n), lambda i: (i, 0)),
        out_shape=jax.ShapeDtypeStruct((m, n), jnp.float32),
    )(a, b)


def _pass_a(ssrc16, sdst16, base16p, src_sp, dst_sp, perm_p, e_pad):
    nblk = e_pad // (_NW * _BP)
    mesh = plsc.VectorSubcoreMesh(core_axis_name="c", subcore_axis_name="s")

    @functools.partial(
        pl.kernel, mesh=mesh,
        out_type=jax.ShapeDtypeStruct((e_pad, 16), jnp.float32),
        compiler_params=pltpu.CompilerParams(use_tc_tiling_on_sc=False),
        scratch_types=[
            pltpu.VMEM((_BP,), jnp.int32),
            pltpu.VMEM((_BP,), jnp.int32),
            pltpu.VMEM((_BP,), jnp.int32),
            pltpu.VMEM((_BP, 16), jnp.float32),
            pltpu.VMEM((_BP, 16), jnp.float32),
            pltpu.VMEM((_BP, 16), jnp.float32),
            pltpu.VMEM((_BP, 16), jnp.float32),
            pltpu.SemaphoreType.DMA,
            pltpu.SemaphoreType.DMA,
            pltpu.SemaphoreType.DMA,
        ],
    )
    def k(ssrc_h, sdst_h, base_h, srcs_h, dsts_h, perm_h, ex2_h,
          si_v, di_v, pi_v, a_v, b_v, c_v, ex_v, sm0, sm1, sm2):
        wid = lax.axis_index("s") * 2 + lax.axis_index("c")

        def blk(kb, carry):
            e0 = pl.multiple_of((wid * nblk + kb) * _BP, 8)
            pltpu.sync_copy(srcs_h.at[pl.ds(e0, _BP)], si_v)
            pltpu.sync_copy(dsts_h.at[pl.ds(e0, _BP)], di_v)
            pltpu.sync_copy(perm_h.at[pl.ds(e0, _BP)], pi_v)
            ca = pltpu.async_copy(ssrc_h.at[si_v], a_v, sm0)
            cb = pltpu.async_copy(sdst_h.at[di_v], b_v, sm1)
            cc = pltpu.async_copy(base_h.at[pi_v], c_v, sm2)
            ca.wait()
            cb.wait()
            cc.wait()

            def body(e, carry):
                al = a_v[e] + b_v[e] + c_v[e]
                al = jnp.where(al >= 0, al, 0.2 * al)
                ex_v[e] = jnp.exp(al)
                return carry
            lax.fori_loop(0, _BP, body, 0)
            pltpu.sync_copy(ex_v, ex2_h.at[pl.ds(e0, _BP)])
            return carry
        lax.fori_loop(0, nblk, blk, 0)

    return k(ssrc16, sdst16, base16p, src_sp, dst_sp, perm_p)


def _pass_b(xt, ea, ex2, src_sp, dst_sp, perm_p, et_sp, row_ptr_p,
            in_ch, n_pad, win, num_types):
    heads = 8
    npt = n_pad // _NW
    nwin = npt // win
    mesh = plsc.VectorSubcoreMesh(core_axis_name="c", subcore_axis_name="s")
    gsz = (win + 1) * heads * in_ch
    fsz = (win + 1) * heads * 16
    tsz = (win + 1) * num_types * 16 + 16
    rp_win = npt + 16

    @functools.partial(
        pl.kernel, mesh=mesh,
        out_type=[
            jax.ShapeDtypeStruct((n_pad * heads * in_ch,), jnp.float32),
            jax.ShapeDtypeStruct((n_pad * heads * 16,), jnp.float32),
            jax.ShapeDtypeStruct((n_pad * num_types * 16,), jnp.float32),
        ],
        compiler_params=pltpu.CompilerParams(use_tc_tiling_on_sc=False),
        scratch_types=[
            pltpu.VMEM((rp_win,), jnp.int32),
            pltpu.VMEM((_BP,), jnp.int32),
            pltpu.VMEM((_BP,), jnp.int32),
            pltpu.VMEM((_BP,), jnp.int32),
            pltpu.VMEM((_BP,), jnp.int32),
            pltpu.VMEM((_BP, in_ch), jnp.float32),
            pltpu.VMEM((_BP, 16), jnp.float32),
            pltpu.VMEM((_BP, 16), jnp.float32),
            pltpu.VMEM((gsz,), jnp.float32),
            pltpu.VMEM((fsz,), jnp.float32),
            pltpu.VMEM((tsz,), jnp.float32),
            pltpu.SemaphoreType.DMA,
            pltpu.SemaphoreType.DMA,
        ],
    )
    def k(xt_h, ea_h, ex2_h, srcs_h, dsts_h, perm_h, ets_h, rp_h,
          g_h, f_h, t_h,
          rp_v, si_v, di_v, pi_v, ei_v, xs_v, ea_v, ex_v,
          gw_v, fw_v, tw_v, sm0, sm1):
        wid = lax.axis_index("s") * 2 + lax.axis_index("c")
        na = pl.multiple_of(wid * npt, 8)
        iota = lax.iota(jnp.int32, 16)
        m8 = iota < 8
        zero16 = jnp.zeros((16,), jnp.float32)
        pltpu.sync_copy(rp_h.at[pl.ds(na, rp_win)], rp_v)

        def wloop(w, wcarry):
            ma = pl.multiple_of(na + w * win, 8)
            woff = pl.multiple_of(w * win, 16)
            fa = rp_v[pl.ds(woff, 16)][0]
            fb = rp_v[pl.ds(woff + win, 16)][0]
            fa_al = fa & (-8)
            nblk = (fb - fa_al + _BP - 1) // _BP

            def zg(i, carry):
                gw_v[pl.ds(i * 16, 16)] = zero16
                return carry
            lax.fori_loop(0, gsz // 16, zg, 0)

            def zf(i, carry):
                fw_v[pl.ds(i * 16, 16)] = zero16
                return carry
            lax.fori_loop(0, fsz // 16, zf, 0)

            def zt(i, carry):
                tw_v[pl.ds(i * 16, 16)] = zero16
                return carry
            lax.fori_loop(0, tsz // 16, zt, 0)

            def blk(kb, carry):
                f0 = pl.multiple_of(fa_al + kb * _BP, 8)
                pltpu.sync_copy(srcs_h.at[pl.ds(f0, _BP)], si_v)
                pltpu.sync_copy(dsts_h.at[pl.ds(f0, _BP)], di_v)
                pltpu.sync_copy(perm_h.at[pl.ds(f0, _BP)], pi_v)
                pltpu.sync_copy(ets_h.at[pl.ds(f0, _BP)], ei_v)
                cx = pltpu.async_copy(xt_h.at[si_v], xs_v, sm0)
                ce = pltpu.async_copy(ea_h.at[pi_v], ea_v, sm1)
                cx.wait()
                ce.wait()
                pltpu.sync_copy(ex2_h.at[pl.ds(f0, _BP)], ex_v)

                def grp(g, c2):
                    dchunk = di_v[pl.ds(g * 16, 16)]
                    echunk = ei_v[pl.ds(g * 16, 16)]
                    for l in range(16):
                        e = g * 16 + l
                        eg = f0 + e
                        valid = (eg >= fa) & (eg < fb)
                        dl = jnp.where(valid, dchunk[l] - ma, win)
                        et = echunk[l]
                        exrow = ex_v[e]
                        earow = ea_v[e]
                        gb = dl * (heads * in_ch)
                        fb2 = dl * (heads * 16)
                        tb = (dl * num_types + et) * 16
                        exm = jnp.where(m8, exrow, 0.0)
                        tw_v[pl.ds(tb, 16)] = tw_v[pl.ds(tb, 16)] + exm

                        def gc(c, c3):
                            xc = xs_v[e, pl.ds(c * 16, 16)]
                            off0 = gb + c * 16
                            for h in range(heads):
                                off = off0 + h * in_ch
                                gw_v[pl.ds(off, 16)] = (
                                    gw_v[pl.ds(off, 16)] + exrow[h] * xc)
                            return c3
                        lax.fori_loop(0, in_ch // 16, gc, 0)

                        for h in range(heads):
                            fo = fb2 + h * 16
                            fw_v[pl.ds(fo, 16)] = (
                                fw_v[pl.ds(fo, 16)] + exrow[h] * earow)
                    return c2
                lax.fori_loop(0, _BP // 16, grp, 0)
                return carry
            lax.fori_loop(0, nblk, blk, 0)

            pltpu.sync_copy(
                gw_v.at[pl.ds(0, win * heads * in_ch)],
                g_h.at[pl.ds(pl.multiple_of(ma * heads * in_ch, 8), win * heads * in_ch)])
            pltpu.sync_copy(
                fw_v.at[pl.ds(0, win * heads * 16)],
                f_h.at[pl.ds(pl.multiple_of(ma * heads * 16, 8), win * heads * 16)])
            pltpu.sync_copy(
                tw_v.at[pl.ds(0, win * num_types * 16)],
                t_h.at[pl.ds(pl.multiple_of(ma * num_types * 16, 8), win * num_types * 16)])
            return wcarry
        lax.fori_loop(0, nwin, wloop, 0)

    return k(xt, ea, ex2, src_sp, dst_sp, perm_p, et_sp, row_ptr_p)


def _layer_sc(x, node_type, edge_attr, edge_type, p, concat, idxs):
    n_nodes = x.shape[0]
    heads, out_ch = p["a_src"].shape
    in_ch = p["W"].shape[0]
    d_edge = p["W_e"].shape[0]
    num_types = p["et_emb"].shape[0]
    src_sp, dst_sp, perm_p, et_sp, row_ptr_p, e_pad, n_pad = idxs

    w3 = p["W"].reshape(in_ch, heads, out_ch)
    we3 = p["W_e"].reshape(d_edge, heads, out_ch)
    et3 = p["et_emb"].reshape(num_types, heads, out_ch)
    a_src_w = jnp.einsum("ihc,hc->ih", w3, p["a_src"])
    a_dst_w = jnp.einsum("ihc,hc->ih", w3, p["a_dst"])
    w_ae = jnp.einsum("dhc,hc->dh", we3, p["a_edge"])
    t_a = jnp.einsum("thc,hc->th", et3, p["a_edge"])

    xt = x + jnp.take(p["nt_emb"], node_type, axis=0)
    ssrc16 = _matmul(xt, jnp.pad(a_src_w, ((0, 0), (0, 8))))
    sdst16 = _matmul(xt, jnp.pad(a_dst_w, ((0, 0), (0, 8))))
    w_ae16 = jnp.pad(w_ae, ((0, 0), (0, 8)))
    t_a16 = jnp.pad(t_a, ((0, 0), (0, 8)))
    base16 = _matmul(edge_attr, w_ae16) + jnp.take(t_a16, edge_type, axis=0)
    base16p = jnp.pad(base16, ((0, e_pad - base16.shape[0]), (0, 0)))

    ex2 = _pass_a(ssrc16, sdst16, base16p, src_sp, dst_sp, perm_p, e_pad)
    win = 32 if in_ch >= 128 else 64
    g_f, f_f, t_f = _pass_b(xt, edge_attr, ex2, src_sp, dst_sp, perm_p,
                            et_sp, row_ptr_p, in_ch, n_pad, win, num_types)
    g3 = g_f.reshape(n_pad, heads, in_ch)[:n_nodes]
    f3 = f_f.reshape(n_pad, heads, 16)[:n_nodes]
    t4 = t_f.reshape(n_pad, num_types, 16)[:n_nodes, :, :heads]
    t3 = jnp.swapaxes(t4, 1, 2)

    dcol = jnp.concatenate([
        jnp.zeros((in_ch + d_edge, 1), jnp.float32),
        jnp.ones((num_types, 1), jnp.float32)], axis=0)
    us, dens = [], []
    for h in range(heads):
        a_h = jnp.concatenate([g3[:, h, :], f3[:, h, :], t3[:, h, :]], axis=1)
        b_h = jnp.concatenate([w3[:, h, :], we3[:, h, :], et3[:, h, :]], axis=0)
        u_h = _matmul(a_h, jnp.concatenate([b_h, dcol], axis=1))
        us.append(u_h[:, :out_ch])
        dens.append(u_h[:, out_ch])
    u = jnp.stack(us, axis=1)
    den = jnp.stack(dens, axis=1)
    out = u / (den[:, :, None] + 1e-16)
    if concat:
        out = out.reshape(n_nodes, heads * out_ch)
    else:
        out = out.mean(axis=1)
    out = out + p["bias"]
    return jnp.where(out > 0, out, jnp.expm1(out))


def kernel(x, edge_index, node_type, edge_attr, edge_type, params1, params2):
    n_nodes = x.shape[0]
    e_edges = edge_index.shape[1]
    src = edge_index[0]
    dst = edge_index[1]
    blk = _NW * _BP
    e_pad = ((e_edges + blk - 1) // blk) * blk
    npt = -(-((n_nodes + _NW - 1) // _NW) // 64) * 64
    n_pad = npt * _NW

    perm = jnp.argsort(dst).astype(jnp.int32)
    dst_s = jnp.take(dst, perm)
    src_s = jnp.take(src, perm)
    et_s = jnp.take(edge_type, perm)
    pad = e_pad - e_edges
    src_sp = jnp.pad(src_s, (0, pad))
    dst_sp = jnp.pad(dst_s, (0, pad))
    perm_p = jnp.pad(perm, (0, pad))
    et_sp = jnp.pad(et_s, (0, pad))
    rp_len = n_pad + npt + 32
    row_ptr_p = jnp.searchsorted(
        dst_s, jnp.arange(rp_len, dtype=jnp.int32), side="left").astype(jnp.int32)
    idxs = (src_sp, dst_sp, perm_p, et_sp, row_ptr_p, e_pad, n_pad)

    h = _layer_sc(x, node_type, edge_attr, edge_type, params1, True, idxs)
    h = _layer_sc(h, node_type, edge_attr, edge_type, params2, False, idxs)
    return h


# speedup vs baseline: 18.2980x; 1.4985x over previous
import functools

import jax
import jax.numpy as jnp
from jax import lax
from jax.experimental import pallas as pl
from jax.experimental.pallas import tpu as pltpu
from jax.experimental.pallas import tpu_sc as plsc

_NW = 32
_BP = 128


def _mm_body(a_ref, b_ref, o_ref):
    o_ref[:] = jnp.dot(a_ref[:], b_ref[:], preferred_element_type=jnp.float32)


def _matmul(a, b):
    m, k = a.shape
    k2, n = b.shape
    assert k == k2
    mb = 2000 if m % 2000 == 0 else m
    grid = m // mb
    return pl.pallas_call(
        _mm_body,
        grid=(grid,),
        in_specs=[
            pl.BlockSpec((mb, k), lambda i: (i, 0)),
            pl.BlockSpec((k, n), lambda i: (0, 0)),
        ],
        out_specs=pl.BlockSpec((mb, n), lambda i: (i, 0)),
        out_shape=jax.ShapeDtypeStruct((m, n), jnp.float32),
    )(a, b)


def _pass_a(ssrc16, sdst16, base16p, src_sp, dst_sp, perm_p, e_pad):
    nblk = e_pad // (_NW * _BP)
    mesh = plsc.VectorSubcoreMesh(core_axis_name="c", subcore_axis_name="s")

    @functools.partial(
        pl.kernel, mesh=mesh,
        out_type=jax.ShapeDtypeStruct((e_pad, 16), jnp.float32),
        compiler_params=pltpu.CompilerParams(use_tc_tiling_on_sc=False),
        scratch_types=[
            pltpu.VMEM((_BP,), jnp.int32),
            pltpu.VMEM((_BP,), jnp.int32),
            pltpu.VMEM((_BP,), jnp.int32),
            pltpu.VMEM((_BP, 16), jnp.float32),
            pltpu.VMEM((_BP, 16), jnp.float32),
            pltpu.VMEM((_BP, 16), jnp.float32),
            pltpu.VMEM((_BP, 16), jnp.float32),
            pltpu.SemaphoreType.DMA,
            pltpu.SemaphoreType.DMA,
            pltpu.SemaphoreType.DMA,
        ],
    )
    def k(ssrc_h, sdst_h, base_h, srcs_h, dsts_h, perm_h, ex2_h,
          si_v, di_v, pi_v, a_v, b_v, c_v, ex_v, sm0, sm1, sm2):
        wid = lax.axis_index("s") * 2 + lax.axis_index("c")

        def blk(kb, carry):
            e0 = pl.multiple_of((wid * nblk + kb) * _BP, 8)
            pltpu.sync_copy(srcs_h.at[pl.ds(e0, _BP)], si_v)
            pltpu.sync_copy(dsts_h.at[pl.ds(e0, _BP)], di_v)
            pltpu.sync_copy(perm_h.at[pl.ds(e0, _BP)], pi_v)
            ca = pltpu.async_copy(ssrc_h.at[si_v], a_v, sm0)
            cb = pltpu.async_copy(sdst_h.at[di_v], b_v, sm1)
            cc = pltpu.async_copy(base_h.at[pi_v], c_v, sm2)
            ca.wait()
            cb.wait()
            cc.wait()

            def body(e, carry):
                al = a_v[e] + b_v[e] + c_v[e]
                al = jnp.where(al >= 0, al, 0.2 * al)
                ex_v[e] = jnp.exp(al)
                return carry
            lax.fori_loop(0, _BP, body, 0)
            pltpu.sync_copy(ex_v, ex2_h.at[pl.ds(e0, _BP)])
            return carry
        lax.fori_loop(0, nblk, blk, 0)

    return k(ssrc16, sdst16, base16p, src_sp, dst_sp, perm_p)


def _pass_b(xt, ea, ex2, src_sp, dst_sp, perm_p, et_sp, row_ptr_p,
            in_ch, n_pad, win, num_types):
    heads = 8
    npt = n_pad // _NW
    nwin = npt // win
    mesh = plsc.VectorSubcoreMesh(core_axis_name="c", subcore_axis_name="s")
    gsz = (win + 1) * heads * in_ch
    fsz = (win + 1) * heads * 16
    tsz = (win + 1) * num_types * 16 + 16
    rp_win = npt + 16

    @functools.partial(
        pl.kernel, mesh=mesh,
        out_type=[
            jax.ShapeDtypeStruct((n_pad * heads * in_ch,), jnp.float32),
            jax.ShapeDtypeStruct((n_pad * heads * 16,), jnp.float32),
            jax.ShapeDtypeStruct((n_pad * num_types * 16,), jnp.float32),
        ],
        compiler_params=pltpu.CompilerParams(use_tc_tiling_on_sc=False),
        scratch_types=[
            pltpu.VMEM((rp_win,), jnp.int32),
            pltpu.VMEM((_BP,), jnp.int32),
            pltpu.VMEM((_BP,), jnp.int32),
            pltpu.VMEM((_BP,), jnp.int32),
            pltpu.VMEM((_BP,), jnp.int32),
            pltpu.VMEM((_BP, in_ch), jnp.float32),
            pltpu.VMEM((_BP, 16), jnp.float32),
            pltpu.VMEM((_BP, 16), jnp.float32),
            pltpu.VMEM((gsz,), jnp.float32),
            pltpu.VMEM((fsz,), jnp.float32),
            pltpu.VMEM((tsz,), jnp.float32),
            pltpu.SemaphoreType.DMA,
            pltpu.SemaphoreType.DMA,
        ],
    )
    def k(xt_h, ea_h, ex2_h, srcs_h, dsts_h, perm_h, ets_h, rp_h,
          g_h, f_h, t_h,
          rp_v, si_v, di_v, pi_v, ei_v, xs_v, ea_v, ex_v,
          gw_v, fw_v, tw_v, sm0, sm1):
        wid = lax.axis_index("s") * 2 + lax.axis_index("c")
        na = pl.multiple_of(wid * npt, 8)
        iota = lax.iota(jnp.int32, 16)
        m8 = iota < 8
        zero16 = jnp.zeros((16,), jnp.float32)
        pltpu.sync_copy(rp_h.at[pl.ds(na, rp_win)], rp_v)

        def wloop(w, wcarry):
            ma = pl.multiple_of(na + w * win, 8)
            woff = pl.multiple_of(w * win, 16)
            fa = rp_v[pl.ds(woff, 16)][0]
            fb = rp_v[pl.ds(woff + win, 16)][0]
            fa_al = fa & (-8)
            nblk = (fb - fa_al + _BP - 1) // _BP

            def zg(i, carry):
                gw_v[pl.ds(i * 16, 16)] = zero16
                return carry
            lax.fori_loop(0, gsz // 16, zg, 0)

            def zf(i, carry):
                fw_v[pl.ds(i * 16, 16)] = zero16
                return carry
            lax.fori_loop(0, fsz // 16, zf, 0)

            def zt(i, carry):
                tw_v[pl.ds(i * 16, 16)] = zero16
                return carry
            lax.fori_loop(0, tsz // 16, zt, 0)

            def blk(kb, carry):
                f0 = pl.multiple_of(fa_al + kb * _BP, 8)
                pltpu.sync_copy(srcs_h.at[pl.ds(f0, _BP)], si_v)
                pltpu.sync_copy(dsts_h.at[pl.ds(f0, _BP)], di_v)
                pltpu.sync_copy(perm_h.at[pl.ds(f0, _BP)], pi_v)
                pltpu.sync_copy(ets_h.at[pl.ds(f0, _BP)], ei_v)
                cx = pltpu.async_copy(xt_h.at[si_v], xs_v, sm0)
                ce = pltpu.async_copy(ea_h.at[pi_v], ea_v, sm1)
                cx.wait()
                ce.wait()
                pltpu.sync_copy(ex2_h.at[pl.ds(f0, _BP)], ex_v)

                def grp(g, c2):
                    dchunk = di_v[pl.ds(g * 16, 16)]
                    echunk = ei_v[pl.ds(g * 16, 16)]
                    for l in range(16):
                        e = g * 16 + l
                        eg = f0 + e
                        valid = (eg >= fa) & (eg < fb)
                        dl = jnp.where(valid, dchunk[l] - ma, win)
                        et = echunk[l]
                        exrow = ex_v[e]
                        earow = ea_v[e]
                        gb = dl * (heads * in_ch)
                        fb2 = dl * (heads * 16)
                        tb = (dl * num_types + et) * 16
                        exm = jnp.where(m8, exrow, 0.0)
                        plsc.addupdate(tw_v.at[pl.ds(tb, 16)], exm)

                        def gc(c, c3):
                            xc = xs_v[e, pl.ds(c * 16, 16)]
                            off0 = gb + c * 16
                            for h in range(heads):
                                off = off0 + h * in_ch
                                plsc.addupdate(
                                    gw_v.at[pl.ds(off, 16)], exrow[h] * xc)
                            return c3
                        lax.fori_loop(0, in_ch // 16, gc, 0)

                        for h in range(heads):
                            fo = fb2 + h * 16
                            plsc.addupdate(
                                fw_v.at[pl.ds(fo, 16)], exrow[h] * earow)
                    return c2
                lax.fori_loop(0, _BP // 16, grp, 0)
                return carry
            lax.fori_loop(0, nblk, blk, 0)

            pltpu.sync_copy(
                gw_v.at[pl.ds(0, win * heads * in_ch)],
                g_h.at[pl.ds(pl.multiple_of(ma * heads * in_ch, 8), win * heads * in_ch)])
            pltpu.sync_copy(
                fw_v.at[pl.ds(0, win * heads * 16)],
                f_h.at[pl.ds(pl.multiple_of(ma * heads * 16, 8), win * heads * 16)])
            pltpu.sync_copy(
                tw_v.at[pl.ds(0, win * num_types * 16)],
                t_h.at[pl.ds(pl.multiple_of(ma * num_types * 16, 8), win * num_types * 16)])
            return wcarry
        lax.fori_loop(0, nwin, wloop, 0)

    return k(xt, ea, ex2, src_sp, dst_sp, perm_p, et_sp, row_ptr_p)


def _layer_sc(x, node_type, edge_attr, edge_type, p, concat, idxs):
    n_nodes = x.shape[0]
    heads, out_ch = p["a_src"].shape
    in_ch = p["W"].shape[0]
    d_edge = p["W_e"].shape[0]
    num_types = p["et_emb"].shape[0]
    src_sp, dst_sp, perm_p, et_sp, row_ptr_p, e_pad, n_pad = idxs

    w3 = p["W"].reshape(in_ch, heads, out_ch)
    we3 = p["W_e"].reshape(d_edge, heads, out_ch)
    et3 = p["et_emb"].reshape(num_types, heads, out_ch)
    a_src_w = jnp.einsum("ihc,hc->ih", w3, p["a_src"])
    a_dst_w = jnp.einsum("ihc,hc->ih", w3, p["a_dst"])
    w_ae = jnp.einsum("dhc,hc->dh", we3, p["a_edge"])
    t_a = jnp.einsum("thc,hc->th", et3, p["a_edge"])

    xt = x + jnp.take(p["nt_emb"], node_type, axis=0)
    ssrc16 = _matmul(xt, jnp.pad(a_src_w, ((0, 0), (0, 8))))
    sdst16 = _matmul(xt, jnp.pad(a_dst_w, ((0, 0), (0, 8))))
    w_ae16 = jnp.pad(w_ae, ((0, 0), (0, 8)))
    t_a16 = jnp.pad(t_a, ((0, 0), (0, 8)))
    base16 = _matmul(edge_attr, w_ae16) + jnp.take(t_a16, edge_type, axis=0)
    base16p = jnp.pad(base16, ((0, e_pad - base16.shape[0]), (0, 0)))

    ex2 = _pass_a(ssrc16, sdst16, base16p, src_sp, dst_sp, perm_p, e_pad)
    win = 32 if in_ch >= 128 else 64
    g_f, f_f, t_f = _pass_b(xt, edge_attr, ex2, src_sp, dst_sp, perm_p,
                            et_sp, row_ptr_p, in_ch, n_pad, win, num_types)
    g3 = g_f.reshape(n_pad, heads, in_ch)[:n_nodes]
    f3 = f_f.reshape(n_pad, heads, 16)[:n_nodes]
    t4 = t_f.reshape(n_pad, num_types, 16)[:n_nodes, :, :heads]
    t3 = jnp.swapaxes(t4, 1, 2)

    dcol = jnp.concatenate([
        jnp.zeros((in_ch + d_edge, 1), jnp.float32),
        jnp.ones((num_types, 1), jnp.float32)], axis=0)
    us, dens = [], []
    for h in range(heads):
        a_h = jnp.concatenate([g3[:, h, :], f3[:, h, :], t3[:, h, :]], axis=1)
        b_h = jnp.concatenate([w3[:, h, :], we3[:, h, :], et3[:, h, :]], axis=0)
        u_h = _matmul(a_h, jnp.concatenate([b_h, dcol], axis=1))
        us.append(u_h[:, :out_ch])
        dens.append(u_h[:, out_ch])
    u = jnp.stack(us, axis=1)
    den = jnp.stack(dens, axis=1)
    out = u / (den[:, :, None] + 1e-16)
    if concat:
        out = out.reshape(n_nodes, heads * out_ch)
    else:
        out = out.mean(axis=1)
    out = out + p["bias"]
    return jnp.where(out > 0, out, jnp.expm1(out))


def kernel(x, edge_index, node_type, edge_attr, edge_type, params1, params2):
    n_nodes = x.shape[0]
    e_edges = edge_index.shape[1]
    src = edge_index[0]
    dst = edge_index[1]
    blk = _NW * _BP
    e_pad = ((e_edges + blk - 1) // blk) * blk
    npt = -(-((n_nodes + _NW - 1) // _NW) // 64) * 64
    n_pad = npt * _NW

    perm = jnp.argsort(dst).astype(jnp.int32)
    dst_s = jnp.take(dst, perm)
    src_s = jnp.take(src, perm)
    et_s = jnp.take(edge_type, perm)
    pad = e_pad - e_edges
    src_sp = jnp.pad(src_s, (0, pad))
    dst_sp = jnp.pad(dst_s, (0, pad))
    perm_p = jnp.pad(perm, (0, pad))
    et_sp = jnp.pad(et_s, (0, pad))
    rp_len = n_pad + npt + 32
    row_ptr_p = jnp.searchsorted(
        dst_s, jnp.arange(rp_len, dtype=jnp.int32), side="left").astype(jnp.int32)
    idxs = (src_sp, dst_sp, perm_p, et_sp, row_ptr_p, e_pad, n_pad)

    h = _layer_sc(x, node_type, edge_attr, edge_type, params1, True, idxs)
    h = _layer_sc(h, node_type, edge_attr, edge_type, params2, False, idxs)
    return h
